# baseline (speedup 1.0000x reference)
import functools

import jax
import jax.numpy as jnp
from jax import lax
from jax.experimental import pallas as pl
from jax.experimental.pallas import tpu as pltpu
from jax.experimental.pallas import tpu_sc as plsc

NUM_USERS = 100000
NUM_ITEMS = 1000000
F = 16
MU = 3.5
L = 2048
NW = 32
PER_W = L // NW
NORM = float(L) ** -0.5

_MESH = plsc.VectorSubcoreMesh(core_axis_name="c", subcore_axis_name="s")
_PARAMS = pltpu.CompilerParams(use_tc_tiling_on_sc=False,
                               needs_layout_passes=False)


def _bcast_lane(vec, j):
    idx = jnp.full((16, 1), j, dtype=jnp.int32)
    dnums = lax.GatherDimensionNumbers(
        offset_dims=(), collapsed_slice_dims=(0,), start_index_map=(0,))
    return lax.gather(vec, idx, dnums, slice_sizes=(1,),
                      mode=lax.GatherScatterMode.PROMISE_IN_BOUNDS)


@functools.partial(
    pl.kernel,
    out_type=(
        jax.ShapeDtypeStruct((NW, F), jnp.float32),
        jax.ShapeDtypeStruct((NW, F), jnp.float32),
    ),
    mesh=_MESH,
    compiler_params=_PARAMS,
    scratch_types=[
        pltpu.VMEM((PER_W,), jnp.int32),
        pltpu.VMEM((PER_W,), jnp.float32),
        pltpu.VMEM((PER_W,), jnp.float32),
        pltpu.VMEM((PER_W, F), jnp.float32),
        pltpu.VMEM((PER_W, F), jnp.float32),
        pltpu.VMEM((F,), jnp.float32),
        pltpu.VMEM((F,), jnp.float32),
        pltpu.SemaphoreType.DMA,
        pltpu.SemaphoreType.DMA,
        pltpu.SemaphoreType.DMA,
    ],
)
def _partials_kernel(items_hbm, ratings_hbm, ib_hbm, x_hbm, y_hbm,
                     out_t_hbm, out_u_hbm,
                     idx_v, rat_v, bias_v, x_v, y_v, t_v, u_v,
                     semx, semy, semb):
    wid = lax.axis_index("s") * 2 + lax.axis_index("c")
    base = wid * PER_W
    pltpu.sync_copy(items_hbm.at[pl.ds(base, PER_W)], idx_v)
    pltpu.sync_copy(ratings_hbm.at[pl.ds(base, PER_W)], rat_v)
    cx = pltpu.async_copy(x_hbm.at[idx_v], x_v, semx)
    cy = pltpu.async_copy(y_hbm.at[idx_v], y_v, semy)
    cb = pltpu.async_copy(ib_hbm.at[idx_v], bias_v, semb)
    cb.wait()
    cx.wait()
    cy.wait()

    t = jnp.zeros((F,), jnp.float32)
    u = jnp.zeros((F,), jnp.float32)
    for c in range(PER_W // 16):
        a_vec = rat_v[pl.ds(c * 16, 16)] - (MU + bias_v[pl.ds(c * 16, 16)])
        for j in range(16):
            ab = _bcast_lane(a_vec, j)
            xr = x_v[c * 16 + j]
            yr = y_v[c * 16 + j]
            t = t + ab * xr + yr
            u = u + xr
    t_v[...] = t
    u_v[...] = u
    pltpu.sync_copy(t_v, out_t_hbm.at[wid])
    pltpu.sync_copy(u_v, out_u_hbm.at[wid])


@functools.partial(
    pl.kernel,
    out_type=jax.ShapeDtypeStruct((F,), jnp.float32),
    mesh=_MESH,
    compiler_params=_PARAMS,
    scratch_types=[
        pltpu.VMEM((16,), jnp.int32),
        pltpu.VMEM((16,), jnp.int32),
        pltpu.VMEM((16,), jnp.float32),
        pltpu.VMEM((16,), jnp.float32),
        pltpu.VMEM((16,), jnp.float32),
        pltpu.VMEM((NW, F), jnp.float32),
        pltpu.VMEM((NW, F), jnp.float32),
        pltpu.VMEM((F,), jnp.float32),
        pltpu.SemaphoreType.DMA,
        pltpu.SemaphoreType.DMA,
        pltpu.SemaphoreType.DMA,
    ],
)
def _combine_kernel(user_hbm, item_hbm, ub_hbm, ib_hbm, qf_hbm,
                    pt_hbm, pu_hbm, out_hbm,
                    usr_v, itm_v, bu_v, bi_v, q_v, pt_v, pu_v, res_v,
                    sem0, sem1, sem2):
    wid = lax.axis_index("s") * 2 + lax.axis_index("c")

    @pl.when(wid == 0)
    def _():
        zero16 = jnp.zeros((16,), jnp.int32)
        cu = pltpu.async_copy(user_hbm.at[zero16], usr_v, sem0)
        ci = pltpu.async_copy(item_hbm.at[zero16], itm_v, sem1)
        cu.wait()
        ci.wait()
        user16 = usr_v[...]
        item16 = itm_v[...]
        qidx = item16 * F + lax.iota(jnp.int32, 16)
        cb = pltpu.async_copy(ub_hbm.at[user16], bu_v, sem0)
        cbi = pltpu.async_copy(ib_hbm.at[item16], bi_v, sem1)
        cq = pltpu.async_copy(qf_hbm.at[qidx], q_v, sem2)
        pltpu.sync_copy(pt_hbm, pt_v)
        pltpu.sync_copy(pu_hbm, pu_v)
        cb.wait()
        cbi.wait()
        cq.wait()

        t = jnp.zeros((F,), jnp.float32)
        u = jnp.zeros((F,), jnp.float32)
        for w in range(NW):
            t = t + pt_v[w]
            u = u + pu_v[w]
        bu16 = bu_v[...]
        bi16 = bi_v[...]
        acc = t - bu16 * u
        s = jnp.sum(q_v[...] * acc) * NORM
        res_v[...] = MU + bu16 + bi16 + s
        pltpu.sync_copy(res_v, out_hbm)


def kernel(user, item, implicit_items, ratings, user_biases, item_biases, Q, X, Y):
    ib_flat = jnp.reshape(item_biases, (-1,))
    ub_flat = jnp.reshape(user_biases, (-1,))
    q_flat = jnp.reshape(Q, (-1,))
    idx = implicit_items.astype(jnp.int32)
    part_t, part_u = _partials_kernel(idx, ratings, ib_flat, X, Y)
    res16 = _combine_kernel(user.astype(jnp.int32), item.astype(jnp.int32),
                            ub_flat, ib_flat, q_flat, part_t, part_u)
    return res16[0:1]


# speedup vs baseline: 13.4457x; 13.4457x over previous
import functools

import jax
import jax.numpy as jnp
from jax import lax
from jax.experimental import pallas as pl
from jax.experimental.pallas import tpu as pltpu
from jax.experimental.pallas import tpu_sc as plsc

F = 16
MU = 3.5
L = 2048
NW = 32
PER_W = L // NW
NORM = float(L) ** -0.5

_MESH = plsc.VectorSubcoreMesh(core_axis_name="c", subcore_axis_name="s")
_PARAMS = pltpu.CompilerParams(needs_layout_passes=False)


def _bcast_lane(vec, j):
    idx = jnp.full((16, 1), j, dtype=jnp.int32)
    dnums = lax.GatherDimensionNumbers(
        offset_dims=(), collapsed_slice_dims=(0,), start_index_map=(0,))
    return lax.gather(vec, idx, dnums, slice_sizes=(1,),
                      mode=lax.GatherScatterMode.PROMISE_IN_BOUNDS)


@functools.partial(
    pl.kernel,
    out_type=(
        jax.ShapeDtypeStruct((NW * F,), jnp.float32),
        jax.ShapeDtypeStruct((NW * F,), jnp.float32),
    ),
    mesh=_MESH,
    compiler_params=_PARAMS,
    scratch_types=[
        pltpu.VMEM((PER_W,), jnp.int32),
        pltpu.VMEM((PER_W,), jnp.float32),
        pltpu.VMEM((PER_W,), jnp.float32),
        pltpu.VMEM((16, F, 128), jnp.float32),
        pltpu.VMEM((16, F, 128), jnp.float32),
        pltpu.VMEM((F,), jnp.float32),
        pltpu.VMEM((F,), jnp.float32),
        pltpu.SemaphoreType.DMA,
        pltpu.SemaphoreType.DMA,
        pltpu.SemaphoreType.DMA,
    ],
)
def _partials_kernel(items_hbm, ratings_hbm, ib_hbm, xt_hbm, yt_hbm,
                     out_t_hbm, out_u_hbm,
                     idx_v, rat_v, bias_v, xbufs, ybufs, t_v, u_v,
                     semx, semy, semb):
    wid = lax.axis_index("s") * 2 + lax.axis_index("c")
    base = wid * PER_W
    pltpu.sync_copy(items_hbm.at[pl.ds(base, PER_W)], idx_v)
    pltpu.sync_copy(ratings_hbm.at[pl.ds(base, PER_W)], rat_v)
    cb = pltpu.async_copy(ib_hbm.at[idx_v], bias_v, semb)

    cb.wait()
    iota16 = lax.iota(jnp.int32, 16)
    t = jnp.zeros((F,), jnp.float32)
    u = jnp.zeros((F,), jnp.float32)
    for c in range(PER_W // 16):
        iv = idx_v[pl.ds(c * 16, 16)]
        csv = (iv >> 7) << 7
        copies = []
        for j in range(16):
            cs = pl.multiple_of(csv[j], 128)
            copies.append(
                pltpu.async_copy(xt_hbm.at[:, pl.ds(cs, 128)], xbufs.at[j],
                                 semx))
            copies.append(
                pltpu.async_copy(yt_hbm.at[:, pl.ds(cs, 128)], ybufs.at[j],
                                 semy))
        for cp in copies:
            cp.wait()
        colc = iv & 127
        a_vec = rat_v[pl.ds(c * 16, 16)] - (MU + bias_v[pl.ds(c * 16, 16)])
        for j in range(16):
            ab = _bcast_lane(a_vec, j)
            colb = _bcast_lane(colc, j)
            row_j = jnp.full((16,), j, dtype=jnp.int32)
            xr = plsc.load_gather(xbufs, [row_j, iota16, colb])
            yr = plsc.load_gather(ybufs, [row_j, iota16, colb])
            t = t + ab * xr + yr
            u = u + xr
    t_v[...] = t
    u_v[...] = u
    pltpu.sync_copy(t_v, out_t_hbm.at[pl.ds(wid * F, F)])
    pltpu.sync_copy(u_v, out_u_hbm.at[pl.ds(wid * F, F)])


@functools.partial(
    pl.kernel,
    out_type=jax.ShapeDtypeStruct((F,), jnp.float32),
    mesh=_MESH,
    compiler_params=_PARAMS,
    scratch_types=[
        pltpu.VMEM((16,), jnp.int32),
        pltpu.VMEM((16,), jnp.int32),
        pltpu.VMEM((16,), jnp.float32),
        pltpu.VMEM((16,), jnp.float32),
        pltpu.VMEM((F, 128), jnp.float32),
        pltpu.VMEM((NW * F,), jnp.float32),
        pltpu.VMEM((NW * F,), jnp.float32),
        pltpu.VMEM((F,), jnp.float32),
        pltpu.SemaphoreType.DMA,
        pltpu.SemaphoreType.DMA,
        pltpu.SemaphoreType.DMA,
    ],
)
def _combine_kernel(user_hbm, item_hbm, ub_hbm, ib_hbm, qt_hbm,
                    pt_hbm, pu_hbm, out_hbm,
                    usr_v, itm_v, bu_v, bi_v, qbuf, pt_v, pu_v, res_v,
                    sem0, sem1, sem2):
    wid = lax.axis_index("s") * 2 + lax.axis_index("c")

    @pl.when(wid == 0)
    def _():
        zero16 = jnp.zeros((16,), jnp.int32)
        cu = pltpu.async_copy(user_hbm.at[zero16], usr_v, sem0)
        ci = pltpu.async_copy(item_hbm.at[zero16], itm_v, sem1)
        cu.wait()
        ci.wait()
        user16 = usr_v[...]
        item16 = itm_v[...]
        item_s = jnp.max(item16)
        cb = pltpu.async_copy(ub_hbm.at[user16], bu_v, sem0)
        cbi = pltpu.async_copy(ib_hbm.at[item16], bi_v, sem1)
        qs = pl.multiple_of((item_s >> 7) << 7, 128)
        cq = pltpu.async_copy(qt_hbm.at[:, pl.ds(qs, 128)], qbuf, sem2)
        pltpu.sync_copy(pt_hbm, pt_v)
        pltpu.sync_copy(pu_hbm, pu_v)
        cb.wait()
        cbi.wait()
        cq.wait()

        iota16 = lax.iota(jnp.int32, 16)
        q16 = plsc.load_gather(qbuf, [iota16, item16 & 127])
        t = jnp.zeros((F,), jnp.float32)
        u = jnp.zeros((F,), jnp.float32)
        for w in range(NW):
            t = t + pt_v[pl.ds(w * F, F)]
            u = u + pu_v[pl.ds(w * F, F)]
        bu16 = bu_v[...]
        bi16 = bi_v[...]
        acc = t - bu16 * u
        s = jnp.sum(q16 * acc) * NORM
        res_v[...] = MU + bu16 + bi16 + s
        pltpu.sync_copy(res_v, out_hbm)


def kernel(user, item, implicit_items, ratings, user_biases, item_biases, Q, X, Y):
    ib_flat = jnp.reshape(item_biases, (-1,))
    ub_flat = jnp.reshape(user_biases, (-1,))
    xt = jnp.transpose(X)
    yt = jnp.transpose(Y)
    qt = jnp.transpose(Q)
    idx = implicit_items.astype(jnp.int32)
    part_t, part_u = _partials_kernel(idx, ratings, ib_flat, xt, yt)
    res16 = _combine_kernel(user.astype(jnp.int32), item.astype(jnp.int32),
                            ub_flat, ib_flat, qt, part_t, part_u)
    return res16[0:1]


# speedup vs baseline: 25.1592x; 1.8712x over previous
import functools

import jax
import jax.numpy as jnp
from jax import lax
from jax.experimental import pallas as pl
from jax.experimental.pallas import tpu as pltpu
from jax.experimental.pallas import tpu_sc as plsc

F = 16
MU = 3.5
L = 2048
NW = 32
PER_W = L // NW
NORM = float(L) ** -0.5

_MESH = plsc.VectorSubcoreMesh(core_axis_name="c", subcore_axis_name="s")
_PARAMS = pltpu.CompilerParams(needs_layout_passes=False)


def _bcast_lane(vec, j):
    idx = jnp.full((16, 1), j, dtype=jnp.int32)
    dnums = lax.GatherDimensionNumbers(
        offset_dims=(), collapsed_slice_dims=(0,), start_index_map=(0,))
    return lax.gather(vec, idx, dnums, slice_sizes=(1,),
                      mode=lax.GatherScatterMode.PROMISE_IN_BOUNDS)


@functools.partial(
    pl.kernel,
    out_type=(
        jax.ShapeDtypeStruct((NW * F,), jnp.float32),
        jax.ShapeDtypeStruct((NW * F,), jnp.float32),
    ),
    mesh=_MESH,
    compiler_params=_PARAMS,
    scratch_types=[
        pltpu.VMEM((PER_W,), jnp.int32),
        pltpu.VMEM((PER_W,), jnp.float32),
        pltpu.VMEM((16, F, 128), jnp.float32),
        pltpu.VMEM((16, F, 128), jnp.float32),
        pltpu.VMEM((16, 1, 128), jnp.float32),
        pltpu.VMEM((F,), jnp.float32),
        pltpu.VMEM((F,), jnp.float32),
        pltpu.SemaphoreType.DMA,
        pltpu.SemaphoreType.DMA,
        pltpu.SemaphoreType.DMA,
    ],
)
def _partials_kernel(items_hbm, ratings_hbm, ibt_hbm, xt_hbm, yt_hbm,
                     out_t_hbm, out_u_hbm,
                     idx_v, rat_v, xbufs, ybufs, bbufs, t_v, u_v,
                     semx, semy, semb):
    wid = lax.axis_index("s") * 2 + lax.axis_index("c")
    base = wid * PER_W
    pltpu.sync_copy(items_hbm.at[pl.ds(base, PER_W)], idx_v)
    pltpu.sync_copy(ratings_hbm.at[pl.ds(base, PER_W)], rat_v)

    iota16 = lax.iota(jnp.int32, 16)
    zero16 = jnp.zeros((16,), jnp.int32)
    t = jnp.zeros((F,), jnp.float32)
    u = jnp.zeros((F,), jnp.float32)
    for c in range(PER_W // 16):
        iv = idx_v[pl.ds(c * 16, 16)]
        csv = (iv >> 7) << 7
        copies = []
        for j in range(16):
            cs = pl.multiple_of(csv[j], 128)
            copies.append(
                pltpu.async_copy(xt_hbm.at[:, pl.ds(cs, 128)], xbufs.at[j],
                                 semx))
            copies.append(
                pltpu.async_copy(yt_hbm.at[:, pl.ds(cs, 128)], ybufs.at[j],
                                 semy))
            copies.append(
                pltpu.async_copy(ibt_hbm.at[:, pl.ds(cs, 128)], bbufs.at[j],
                                 semb))
        for cp in copies:
            cp.wait()
        colc = iv & 127
        bias_c = plsc.load_gather(bbufs, [iota16, zero16, colc])
        a_vec = rat_v[pl.ds(c * 16, 16)] - (MU + bias_c)
        for j in range(16):
            ab = _bcast_lane(a_vec, j)
            colb = _bcast_lane(colc, j)
            row_j = jnp.full((16,), j, dtype=jnp.int32)
            xr = plsc.load_gather(xbufs, [row_j, iota16, colb])
            yr = plsc.load_gather(ybufs, [row_j, iota16, colb])
            t = t + ab * xr + yr
            u = u + xr
    t_v[...] = t
    u_v[...] = u
    pltpu.sync_copy(t_v, out_t_hbm.at[pl.ds(wid * F, F)])
    pltpu.sync_copy(u_v, out_u_hbm.at[pl.ds(wid * F, F)])


@functools.partial(
    pl.kernel,
    out_type=jax.ShapeDtypeStruct((F,), jnp.float32),
    mesh=_MESH,
    compiler_params=_PARAMS,
    scratch_types=[
        pltpu.VMEM((16,), jnp.int32),
        pltpu.VMEM((16,), jnp.int32),
        pltpu.VMEM((1, 128), jnp.float32),
        pltpu.VMEM((1, 128), jnp.float32),
        pltpu.VMEM((F, 128), jnp.float32),
        pltpu.VMEM((NW * F,), jnp.float32),
        pltpu.VMEM((NW * F,), jnp.float32),
        pltpu.VMEM((F,), jnp.float32),
        pltpu.SemaphoreType.DMA,
        pltpu.SemaphoreType.DMA,
        pltpu.SemaphoreType.DMA,
    ],
)
def _combine_kernel(user_hbm, item_hbm, ubt_hbm, ibt_hbm, qt_hbm,
                    pt_hbm, pu_hbm, out_hbm,
                    usr_v, itm_v, ubuf, ibuf, qbuf, pt_v, pu_v, res_v,
                    sem0, sem1, sem2):
    wid = lax.axis_index("s") * 2 + lax.axis_index("c")

    @pl.when(wid == 0)
    def _():
        zero16 = jnp.zeros((16,), jnp.int32)
        cu = pltpu.async_copy(user_hbm.at[zero16], usr_v, sem0)
        ci = pltpu.async_copy(item_hbm.at[zero16], itm_v, sem1)
        cu.wait()
        ci.wait()
        user16 = usr_v[...]
        item16 = itm_v[...]
        user_s = jnp.max(user16)
        item_s = jnp.max(item16)
        us = pl.multiple_of((user_s >> 7) << 7, 128)
        cb = pltpu.async_copy(ubt_hbm.at[:, pl.ds(us, 128)], ubuf, sem0)
        bs = pl.multiple_of((item_s >> 7) << 7, 128)
        cbi = pltpu.async_copy(ibt_hbm.at[:, pl.ds(bs, 128)], ibuf, sem1)
        qs = pl.multiple_of((item_s >> 7) << 7, 128)
        cq = pltpu.async_copy(qt_hbm.at[:, pl.ds(qs, 128)], qbuf, sem2)
        pltpu.sync_copy(pt_hbm, pt_v)
        pltpu.sync_copy(pu_hbm, pu_v)
        cb.wait()
        cbi.wait()
        cq.wait()

        iota16 = lax.iota(jnp.int32, 16)
        colu = user16 & 127
        coli = item16 & 127
        bu16 = plsc.load_gather(ubuf, [zero16, colu])
        bi16 = plsc.load_gather(ibuf, [zero16, coli])
        q16 = plsc.load_gather(qbuf, [iota16, coli])
        t = jnp.zeros((F,), jnp.float32)
        u = jnp.zeros((F,), jnp.float32)
        for w in range(NW):
            t = t + pt_v[pl.ds(w * F, F)]
            u = u + pu_v[pl.ds(w * F, F)]
        acc = t - bu16 * u
        s = jnp.sum(q16 * acc) * NORM
        res_v[...] = MU + bu16 + bi16 + s
        pltpu.sync_copy(res_v, out_hbm)


def kernel(user, item, implicit_items, ratings, user_biases, item_biases, Q, X, Y):
    ibt = jnp.transpose(item_biases)
    ubt = jnp.transpose(user_biases)
    xt = jnp.transpose(X)
    yt = jnp.transpose(Y)
    qt = jnp.transpose(Q)
    idx = implicit_items.astype(jnp.int32)
    part_t, part_u = _partials_kernel(idx, ratings, ibt, xt, yt)
    res16 = _combine_kernel(user.astype(jnp.int32), item.astype(jnp.int32),
                            ubt, ibt, qt, part_t, part_u)
    return res16[0:1]


# speedup vs baseline: 26.4331x; 1.0506x over previous
import functools

import jax
import jax.numpy as jnp
from jax import lax
from jax.experimental import pallas as pl
from jax.experimental.pallas import tpu as pltpu
from jax.experimental.pallas import tpu_sc as plsc

F = 16
MU = 3.5
L = 2048
NW = 32
PER_W = L // NW
NORM = float(L) ** -0.5

_MESH = plsc.VectorSubcoreMesh(core_axis_name="c", subcore_axis_name="s")
_PARAMS = pltpu.CompilerParams(needs_layout_passes=False)


def _bcast_lane(vec, j):
    idx = jnp.full((16, 1), j, dtype=jnp.int32)
    dnums = lax.GatherDimensionNumbers(
        offset_dims=(), collapsed_slice_dims=(0,), start_index_map=(0,))
    return lax.gather(vec, idx, dnums, slice_sizes=(1,),
                      mode=lax.GatherScatterMode.PROMISE_IN_BOUNDS)


@functools.partial(
    pl.kernel,
    out_type=jax.ShapeDtypeStruct((NW * F,), jnp.float32),
    mesh=_MESH,
    compiler_params=_PARAMS,
    scratch_types=[
        pltpu.VMEM((PER_W,), jnp.int32),
        pltpu.VMEM((PER_W,), jnp.float32),
        pltpu.VMEM((16, F, 128), jnp.float32),
        pltpu.VMEM((16, F, 128), jnp.float32),
        pltpu.VMEM((16, 1, 128), jnp.float32),
        pltpu.VMEM((16,), jnp.int32),
        pltpu.VMEM((16,), jnp.int32),
        pltpu.VMEM((F, 128), jnp.float32),
        pltpu.VMEM((1, 128), jnp.float32),
        pltpu.VMEM((1, 128), jnp.float32),
        pltpu.VMEM((F,), jnp.float32),
        pltpu.SemaphoreType.DMA,
        pltpu.SemaphoreType.DMA,
        pltpu.SemaphoreType.DMA,
        pltpu.SemaphoreType.DMA,
    ],
)
def _sc_kernel(items_hbm, ratings_hbm, user_hbm, item_hbm,
               ibt_hbm, ubt_hbm, xt_hbm, yt_hbm, qt_hbm,
               out_hbm,
               idx_v, rat_v, xbufs, ybufs, bbufs, itm_v, usr_v,
               qbuf, ubuf, ibbuf, row_v,
               semx, semy, semb, semq):
    wid = lax.axis_index("s") * 2 + lax.axis_index("c")
    base = wid * PER_W
    zero16 = jnp.zeros((16,), jnp.int32)
    iota16 = lax.iota(jnp.int32, 16)

    ci = pltpu.async_copy(item_hbm.at[zero16], itm_v, semq)
    pltpu.sync_copy(items_hbm.at[pl.ds(base, PER_W)], idx_v)
    pltpu.sync_copy(ratings_hbm.at[pl.ds(base, PER_W)], rat_v)
    ci.wait()
    item16 = itm_v[...]
    item_s = jnp.max(item16)
    qs = pl.multiple_of((item_s >> 7) << 7, 128)
    cq = pltpu.async_copy(qt_hbm.at[:, pl.ds(qs, 128)], qbuf, semq)

    t = jnp.zeros((F,), jnp.float32)
    u = jnp.zeros((F,), jnp.float32)
    for c in range(PER_W // 16):
        iv = idx_v[pl.ds(c * 16, 16)]
        csv = (iv >> 7) << 7
        copies = []
        for j in range(16):
            cs = pl.multiple_of(csv[j], 128)
            copies.append(
                pltpu.async_copy(xt_hbm.at[:, pl.ds(cs, 128)], xbufs.at[j],
                                 semx))
            copies.append(
                pltpu.async_copy(yt_hbm.at[:, pl.ds(cs, 128)], ybufs.at[j],
                                 semy))
            copies.append(
                pltpu.async_copy(ibt_hbm.at[:, pl.ds(cs, 128)], bbufs.at[j],
                                 semb))
        for cp in copies:
            cp.wait()
        colc = iv & 127
        bias_c = plsc.load_gather(bbufs, [iota16, zero16, colc])
        a_vec = rat_v[pl.ds(c * 16, 16)] - (MU + bias_c)
        for j in range(16):
            ab = _bcast_lane(a_vec, j)
            colb = _bcast_lane(colc, j)
            row_j = jnp.full((16,), j, dtype=jnp.int32)
            xr = plsc.load_gather(xbufs, [row_j, iota16, colb])
            yr = plsc.load_gather(ybufs, [row_j, iota16, colb])
            t = t + ab * xr + yr
            u = u + xr
    cq.wait()
    q16 = plsc.load_gather(qbuf, [iota16, item16 & 127])
    s_w = jnp.sum(q16 * t)
    v_w = jnp.sum(q16 * u)
    row = jnp.where(iota16 == 0, s_w, 0.0) + jnp.where(iota16 == 1, v_w, 0.0)

    @pl.when(wid == 0)
    def _():
        cu = pltpu.async_copy(user_hbm.at[zero16], usr_v, semq)
        cu.wait()
        user16 = usr_v[...]
        user_s = jnp.max(user16)
        us = pl.multiple_of((user_s >> 7) << 7, 128)
        cub = pltpu.async_copy(ubt_hbm.at[:, pl.ds(us, 128)], ubuf, semq)
        ib_s = pl.multiple_of((item_s >> 7) << 7, 128)
        cib = pltpu.async_copy(ibt_hbm.at[:, pl.ds(ib_s, 128)], ibbuf, semb)
        cub.wait()
        cib.wait()
        bu16 = plsc.load_gather(ubuf, [zero16, user16 & 127])
        bi16 = plsc.load_gather(ibbuf, [zero16, item16 & 127])
        row_v[...] = (row + jnp.where(iota16 == 2, bu16, 0.0)
                      + jnp.where(iota16 == 3, bi16, 0.0))

    @pl.when(wid != 0)
    def _():
        row_v[...] = row

    pltpu.sync_copy(row_v, out_hbm.at[pl.ds(wid * F, F)])


def _tc_combine(parts_ref, out_ref):
    parts = parts_ref[...]
    i = lax.iota(jnp.int32, NW * F)
    lane = i % 16
    ss = jnp.sum(jnp.where(lane == 0, parts, 0.0))
    sv = jnp.sum(jnp.where(lane == 1, parts, 0.0))
    bu = jnp.sum(jnp.where(i == 2, parts, 0.0))
    bi = jnp.sum(jnp.where(i == 3, parts, 0.0))
    rui = MU + bu + bi + NORM * (ss - bu * sv)
    out_ref[...] = jnp.full((1, 128), rui, dtype=jnp.float32)


def kernel(user, item, implicit_items, ratings, user_biases, item_biases, Q, X, Y):
    ibt = jnp.transpose(item_biases)
    ubt = jnp.transpose(user_biases)
    xt = jnp.transpose(X)
    yt = jnp.transpose(Y)
    qt = jnp.transpose(Q)
    idx = implicit_items.astype(jnp.int32)
    parts = _sc_kernel(idx, ratings, user.astype(jnp.int32),
                       item.astype(jnp.int32), ibt, ubt, xt, yt, qt)
    rui = pl.pallas_call(
        _tc_combine,
        out_shape=jax.ShapeDtypeStruct((1, 128), jnp.float32),
    )(parts)
    return rui[0, 0:1]


# speedup vs baseline: 28.1808x; 1.0661x over previous
import functools

import jax
import jax.numpy as jnp
from jax import lax
from jax.experimental import pallas as pl
from jax.experimental.pallas import tpu as pltpu
from jax.experimental.pallas import tpu_sc as plsc

F = 16
MU = 3.5
L = 2048
NW = 32
PER_W = L // NW
NORM = float(L) ** -0.5

_MESH = plsc.VectorSubcoreMesh(core_axis_name="c", subcore_axis_name="s")
_PARAMS = pltpu.CompilerParams(needs_layout_passes=False)


def _bcast_lane(vec, j):
    idx = jnp.full((16, 1), j, dtype=jnp.int32)
    dnums = lax.GatherDimensionNumbers(
        offset_dims=(), collapsed_slice_dims=(0,), start_index_map=(0,))
    return lax.gather(vec, idx, dnums, slice_sizes=(1,),
                      mode=lax.GatherScatterMode.PROMISE_IN_BOUNDS)


@functools.partial(
    pl.kernel,
    out_type=jax.ShapeDtypeStruct((NW * F,), jnp.float32),
    mesh=_MESH,
    compiler_params=_PARAMS,
    scratch_types=[
        pltpu.VMEM((PER_W + 16,), jnp.int32),
        pltpu.VMEM((PER_W + 16,), jnp.float32),
        pltpu.VMEM((2, 8, F, 128), jnp.float32),
        pltpu.VMEM((2, 8, F, 128), jnp.float32),
        pltpu.VMEM((2, 8, 1, 128), jnp.float32),
        pltpu.VMEM((16,), jnp.int32),
        pltpu.VMEM((16,), jnp.int32),
        pltpu.VMEM((F, 128), jnp.float32),
        pltpu.VMEM((1, 128), jnp.float32),
        pltpu.VMEM((1, 128), jnp.float32),
        pltpu.VMEM((F,), jnp.float32),
        pltpu.SemaphoreType.DMA,
        pltpu.SemaphoreType.DMA,
        pltpu.SemaphoreType.DMA,
        pltpu.SemaphoreType.DMA,
    ],
)
def _sc_kernel(items_hbm, ratings_hbm, user_hbm, item_hbm,
               ibt_hbm, ubt_hbm, xt_hbm, yt_hbm, qt_hbm,
               out_hbm,
               idx_v, rat_v, xbufs, ybufs, bbufs, itm_v, usr_v,
               qbuf, ubuf, ibbuf, row_v,
               semx, semy, semb, semq):
    wid = lax.axis_index("s") * 2 + lax.axis_index("c")
    base = wid * PER_W
    zero16 = jnp.zeros((16,), jnp.int32)
    iota16 = lax.iota(jnp.int32, 16)

    ci = pltpu.async_copy(item_hbm.at[zero16], itm_v, semq)
    pltpu.sync_copy(items_hbm.at[pl.ds(base, PER_W)], idx_v.at[pl.ds(0, PER_W)])
    pltpu.sync_copy(ratings_hbm.at[pl.ds(base, PER_W)], rat_v.at[pl.ds(0, PER_W)])
    ci.wait()
    item16 = itm_v[...]
    item_s = jnp.max(item16)
    qs = pl.multiple_of((item_s >> 7) << 7, 128)
    cq = pltpu.async_copy(qt_hbm.at[:, pl.ds(qs, 128)], qbuf, semq)

    t = jnp.zeros((F,), jnp.float32)
    u = jnp.zeros((F,), jnp.float32)
    NWAVE = PER_W // 8
    rows8 = iota16 & 7

    def _issue(w):
        iv = idx_v[pl.ds(w * 8, 16)]
        csv = (iv >> 7) << 7
        b = w % 2
        cps = []
        for j in range(8):
            cs = pl.multiple_of(csv[j], 128)
            cps.append(
                pltpu.async_copy(xt_hbm.at[:, pl.ds(cs, 128)],
                                 xbufs.at[b, j], semx))
            cps.append(
                pltpu.async_copy(yt_hbm.at[:, pl.ds(cs, 128)],
                                 ybufs.at[b, j], semy))
            cps.append(
                pltpu.async_copy(ibt_hbm.at[:, pl.ds(cs, 128)],
                                 bbufs.at[b, j], semb))
        return cps

    inflight = {0: _issue(0), 1: _issue(1)}
    for w in range(NWAVE):
        for cp in inflight.pop(w):
            cp.wait()
        b = w % 2
        bsel = jnp.full((16,), b, dtype=jnp.int32)
        iv = idx_v[pl.ds(w * 8, 16)]
        colc = iv & 127
        bias_w = plsc.load_gather(bbufs, [bsel, rows8, zero16, colc])
        a_w = rat_v[pl.ds(w * 8, 16)] - (MU + bias_w)
        for j in range(8):
            ab = _bcast_lane(a_w, j)
            colb = _bcast_lane(colc, j)
            row_j = jnp.full((16,), j, dtype=jnp.int32)
            xr = plsc.load_gather(xbufs, [bsel, row_j, iota16, colb])
            yr = plsc.load_gather(ybufs, [bsel, row_j, iota16, colb])
            t = t + ab * xr + yr
            u = u + xr
        if w + 2 < NWAVE:
            inflight[w + 2] = _issue(w + 2)
    cq.wait()
    q16 = plsc.load_gather(qbuf, [iota16, item16 & 127])
    s_w = jnp.sum(q16 * t)
    v_w = jnp.sum(q16 * u)
    row = jnp.where(iota16 == 0, s_w, 0.0) + jnp.where(iota16 == 1, v_w, 0.0)

    @pl.when(wid == 0)
    def _():
        cu = pltpu.async_copy(user_hbm.at[zero16], usr_v, semq)
        cu.wait()
        user16 = usr_v[...]
        user_s = jnp.max(user16)
        us = pl.multiple_of((user_s >> 7) << 7, 128)
        cub = pltpu.async_copy(ubt_hbm.at[:, pl.ds(us, 128)], ubuf, semq)
        ib_s = pl.multiple_of((item_s >> 7) << 7, 128)
        cib = pltpu.async_copy(ibt_hbm.at[:, pl.ds(ib_s, 128)], ibbuf, semb)
        cub.wait()
        cib.wait()
        bu16 = plsc.load_gather(ubuf, [zero16, user16 & 127])
        bi16 = plsc.load_gather(ibbuf, [zero16, item16 & 127])
        row_v[...] = (row + jnp.where(iota16 == 2, bu16, 0.0)
                      + jnp.where(iota16 == 3, bi16, 0.0))

    @pl.when(wid != 0)
    def _():
        row_v[...] = row

    pltpu.sync_copy(row_v, out_hbm.at[pl.ds(wid * F, F)])


def _tc_combine(parts_ref, out_ref):
    parts = parts_ref[...]
    i = lax.iota(jnp.int32, NW * F)
    lane = i % 16
    ss = jnp.sum(jnp.where(lane == 0, parts, 0.0))
    sv = jnp.sum(jnp.where(lane == 1, parts, 0.0))
    bu = jnp.sum(jnp.where(i == 2, parts, 0.0))
    bi = jnp.sum(jnp.where(i == 3, parts, 0.0))
    rui = MU + bu + bi + NORM * (ss - bu * sv)
    out_ref[...] = jnp.full((1, 128), rui, dtype=jnp.float32)


def kernel(user, item, implicit_items, ratings, user_biases, item_biases, Q, X, Y):
    ibt = jnp.transpose(item_biases)
    ubt = jnp.transpose(user_biases)
    xt = jnp.transpose(X)
    yt = jnp.transpose(Y)
    qt = jnp.transpose(Q)
    idx = implicit_items.astype(jnp.int32)
    parts = _sc_kernel(idx, ratings, user.astype(jnp.int32),
                       item.astype(jnp.int32), ibt, ubt, xt, yt, qt)
    rui = pl.pallas_call(
        _tc_combine,
        out_shape=jax.ShapeDtypeStruct((1, 128), jnp.float32),
    )(parts)
    return rui[0, 0:1]


# speedup vs baseline: 28.2668x; 1.0031x over previous
import functools

import jax
import jax.numpy as jnp
from jax import lax
from jax.experimental import pallas as pl
from jax.experimental.pallas import tpu as pltpu
from jax.experimental.pallas import tpu_sc as plsc

F = 16
MU = 3.5
L = 2048
NW = 32
PER_W = L // NW
NORM = float(L) ** -0.5

_MESH = plsc.VectorSubcoreMesh(core_axis_name="c", subcore_axis_name="s")
_PARAMS = pltpu.CompilerParams(needs_layout_passes=False)


def _bcast_lane(vec, j):
    idx = jnp.full((16, 1), j, dtype=jnp.int32)
    dnums = lax.GatherDimensionNumbers(
        offset_dims=(), collapsed_slice_dims=(0,), start_index_map=(0,))
    return lax.gather(vec, idx, dnums, slice_sizes=(1,),
                      mode=lax.GatherScatterMode.PROMISE_IN_BOUNDS)


@functools.partial(
    pl.kernel,
    out_type=jax.ShapeDtypeStruct((NW * F,), jnp.float32),
    mesh=_MESH,
    compiler_params=_PARAMS,
    scratch_types=[
        pltpu.VMEM((PER_W + 16,), jnp.int32),
        pltpu.VMEM((PER_W + 16,), jnp.float32),
        pltpu.VMEM((3, 8, F, 128), jnp.float32),
        pltpu.VMEM((3, 8, F, 128), jnp.float32),
        pltpu.VMEM((3, 8, 1, 128), jnp.float32),
        pltpu.VMEM((16,), jnp.int32),
        pltpu.VMEM((16,), jnp.int32),
        pltpu.VMEM((F, 128), jnp.float32),
        pltpu.VMEM((1, 128), jnp.float32),
        pltpu.VMEM((1, 128), jnp.float32),
        pltpu.VMEM((F,), jnp.float32),
        pltpu.SemaphoreType.DMA,
        pltpu.SemaphoreType.DMA,
        pltpu.SemaphoreType.DMA,
        pltpu.SemaphoreType.DMA,
    ],
)
def _sc_kernel(items_hbm, ratings_hbm, user_hbm, item_hbm,
               ibt_hbm, ubt_hbm, xt_hbm, yt_hbm, qt_hbm,
               out_hbm,
               idx_v, rat_v, xbufs, ybufs, bbufs, itm_v, usr_v,
               qbuf, ubuf, ibbuf, row_v,
               semx, semy, semb, semq):
    wid = lax.axis_index("s") * 2 + lax.axis_index("c")
    base = wid * PER_W
    zero16 = jnp.zeros((16,), jnp.int32)
    iota16 = lax.iota(jnp.int32, 16)

    ci = pltpu.async_copy(item_hbm.at[zero16], itm_v, semq)
    pltpu.sync_copy(items_hbm.at[pl.ds(base, PER_W)], idx_v.at[pl.ds(0, PER_W)])
    pltpu.sync_copy(ratings_hbm.at[pl.ds(base, PER_W)], rat_v.at[pl.ds(0, PER_W)])
    ci.wait()
    item16 = itm_v[...]
    item_s = jnp.max(item16)
    qs = pl.multiple_of((item_s >> 7) << 7, 128)
    cq = pltpu.async_copy(qt_hbm.at[:, pl.ds(qs, 128)], qbuf, semq)

    t = jnp.zeros((F,), jnp.float32)
    u = jnp.zeros((F,), jnp.float32)
    NWAVE = PER_W // 8
    rows8 = iota16 & 7

    def _issue(w):
        iv = idx_v[pl.ds(w * 8, 16)]
        csv = (iv >> 7) << 7
        b = w % 3
        cps = []
        for j in range(8):
            cs = pl.multiple_of(csv[j], 128)
            cps.append(
                pltpu.async_copy(xt_hbm.at[:, pl.ds(cs, 128)],
                                 xbufs.at[b, j], semx))
            cps.append(
                pltpu.async_copy(yt_hbm.at[:, pl.ds(cs, 128)],
                                 ybufs.at[b, j], semy))
            cps.append(
                pltpu.async_copy(ibt_hbm.at[:, pl.ds(cs, 128)],
                                 bbufs.at[b, j], semb))
        return cps

    inflight = {0: _issue(0), 1: _issue(1), 2: _issue(2)}
    for w in range(NWAVE):
        for cp in inflight.pop(w):
            cp.wait()
        b = w % 3
        bsel = jnp.full((16,), b, dtype=jnp.int32)
        iv = idx_v[pl.ds(w * 8, 16)]
        colc = iv & 127
        bias_w = plsc.load_gather(bbufs, [bsel, rows8, zero16, colc])
        a_w = rat_v[pl.ds(w * 8, 16)] - (MU + bias_w)
        for j in range(8):
            ab = _bcast_lane(a_w, j)
            colb = _bcast_lane(colc, j)
            row_j = jnp.full((16,), j, dtype=jnp.int32)
            xr = plsc.load_gather(xbufs, [bsel, row_j, iota16, colb])
            yr = plsc.load_gather(ybufs, [bsel, row_j, iota16, colb])
            t = t + ab * xr + yr
            u = u + xr
        if w + 3 < NWAVE:
            inflight[w + 3] = _issue(w + 3)
    cq.wait()
    q16 = plsc.load_gather(qbuf, [iota16, item16 & 127])
    s_w = jnp.sum(q16 * t)
    v_w = jnp.sum(q16 * u)
    row = jnp.where(iota16 == 0, s_w, 0.0) + jnp.where(iota16 == 1, v_w, 0.0)

    @pl.when(wid == 0)
    def _():
        cu = pltpu.async_copy(user_hbm.at[zero16], usr_v, semq)
        cu.wait()
        user16 = usr_v[...]
        user_s = jnp.max(user16)
        us = pl.multiple_of((user_s >> 7) << 7, 128)
        cub = pltpu.async_copy(ubt_hbm.at[:, pl.ds(us, 128)], ubuf, semq)
        ib_s = pl.multiple_of((item_s >> 7) << 7, 128)
        cib = pltpu.async_copy(ibt_hbm.at[:, pl.ds(ib_s, 128)], ibbuf, semb)
        cub.wait()
        cib.wait()
        bu16 = plsc.load_gather(ubuf, [zero16, user16 & 127])
        bi16 = plsc.load_gather(ibbuf, [zero16, item16 & 127])
        row_v[...] = (row + jnp.where(iota16 == 2, bu16, 0.0)
                      + jnp.where(iota16 == 3, bi16, 0.0))

    @pl.when(wid != 0)
    def _():
        row_v[...] = row

    pltpu.sync_copy(row_v, out_hbm.at[pl.ds(wid * F, F)])


def _tc_combine(parts_ref, out_ref):
    parts = parts_ref[...]
    i = lax.iota(jnp.int32, NW * F)
    lane = i % 16
    ss = jnp.sum(jnp.where(lane == 0, parts, 0.0))
    sv = jnp.sum(jnp.where(lane == 1, parts, 0.0))
    bu = jnp.sum(jnp.where(i == 2, parts, 0.0))
    bi = jnp.sum(jnp.where(i == 3, parts, 0.0))
    rui = MU + bu + bi + NORM * (ss - bu * sv)
    out_ref[...] = jnp.full((1, 128), rui, dtype=jnp.float32)


def kernel(user, item, implicit_items, ratings, user_biases, item_biases, Q, X, Y):
    ibt = jnp.transpose(item_biases)
    ubt = jnp.transpose(user_biases)
    xt = jnp.transpose(X)
    yt = jnp.transpose(Y)
    qt = jnp.transpose(Q)
    idx = implicit_items.astype(jnp.int32)
    parts = _sc_kernel(idx, ratings, user.astype(jnp.int32),
                       item.astype(jnp.int32), ibt, ubt, xt, yt, qt)
    rui = pl.pallas_call(
        _tc_combine,
        out_shape=jax.ShapeDtypeStruct((1, 128), jnp.float32),
    )(parts)
    return rui[0, 0:1]
